# baseline (speedup 1.0000x reference)
import functools

import jax
import jax.numpy as jnp
import numpy as np
from jax import lax
from jax.experimental import pallas as pl
from jax.experimental.pallas import tpu as pltpu
from jax.experimental.pallas import tpu_sc as plsc

N = 10000
D = 128
HD = D // 2
E = 50000
NUM_ET = 10

_SRC = [1, 1, 1, 0, 0, 0, 2, 0, 1, 2]
_DST = [0, 0, 0, 1, 1, 1, 2, 0, 2, 0]

_ET16 = list(range(10)) + list(range(6))
_SRC16 = [_SRC[e] for e in _ET16]
_DST16 = [_DST[e] for e in _ET16]

N_PAD = 10240
E_PAD = 51200
CH = 128
CHUNKS = E_PAD // CH
NSUB = 16
NCORE = 2
CPS = CHUNKS // NSUB
ROWS_PER_SUB = N_PAD // NSUB

_mesh = plsc.VectorSubcoreMesh(core_axis_name="c", subcore_axis_name="s",
                               num_cores=NCORE, num_subcores=NSUB)


def _mm_body(x_ref, wl_ref, wr_ref, o_ref):
    x = x_ref[0]
    o_ref[0, 0] = jnp.dot(x, wl_ref[0], preferred_element_type=jnp.float32)
    o_ref[1, 0] = jnp.dot(x, wr_ref[0], preferred_element_type=jnp.float32)


def _per_etype_matmul(x3, wl, wr):
    bn = 512
    return pl.pallas_call(
        _mm_body,
        grid=(NUM_ET, N_PAD // bn),
        in_specs=[
            pl.BlockSpec((1, bn, D),
                         lambda e, i: (jnp.where(
                             (e < 3) | (e == 8), 1,
                             jnp.where((e == 6) | (e == 9), 2, 0)), i, 0)),
            pl.BlockSpec((1, D, HD), lambda e, i: (e, 0, 0)),
            pl.BlockSpec((1, D, HD), lambda e, i: (e, 0, 0)),
        ],
        out_specs=pl.BlockSpec((2, 1, bn, HD), lambda e, i: (0, e, i, 0)),
        out_shape=jax.ShapeDtypeStruct((2, NUM_ET, N_PAD, HD), jnp.float32),
    )(x3, wl, wr)


def _segsum_body(with_counts, wh_hbm, src_hbm, dst_hbm, *refs):
    if with_counts:
        (sums_hbm, cnts_hbm, src_v, dst_v, rows0_v, rows1_v, rows2_v, ones_v,
         zrows_v, zcnt_v, acc_sh, cnt_sh, g0, g1, g2, s0, s1, s2, zsem) = refs
    else:
        (sums_hbm, src_v, dst_v, rows0_v, rows1_v, rows2_v, zrows_v, acc_sh,
         g0, g1, g2, s0, s1, s2, zsem) = refs
    rows = (rows0_v, rows1_v, rows2_v)
    sems = (g0, g1, g2)
    ssems = (s0, s1, s2)
    c = lax.axis_index("c")
    w = lax.axis_index("s")
    zero16 = jnp.zeros((16,), jnp.float32)
    one16 = jnp.ones((16,), jnp.float32)

    @pl.loop(0, CH)
    def _(r):
        for k in range(HD // 16):
            zrows_v[r, pl.ds(16 * k, 16)] = zero16
        if with_counts:
            zcnt_v[r, pl.ds(0, 16)] = zero16
            ones_v[r, pl.ds(0, 16)] = one16

    def start_gather(k, b):
        pltpu.async_copy(wh_hbm.at[src_v.at[k]], rows[b], sems[b])

    def wait_gather(k, b):
        pltpu.make_async_copy(wh_hbm.at[src_v.at[k]], rows[b], sems[b]).wait()

    def start_scatter(e, k, b):
        pltpu.async_copy(rows[b], acc_sh.at[dst_v.at[k]], ssems[b], add=True)
        if with_counts:
            @pl.when(c == e // 5)
            def _():
                pltpu.async_copy(ones_v, cnt_sh.at[dst_v.at[k]], ssems[b],
                                 add=True)

    def wait_scatter(e, k, b):
        pltpu.make_async_copy(rows[b], acc_sh.at[dst_v.at[k]],
                              ssems[b]).wait()
        if with_counts:
            @pl.when(c == e // 5)
            def _():
                pltpu.make_async_copy(ones_v, cnt_sh.at[dst_v.at[k]],
                                      ssems[b]).wait()

    @pl.loop(0, NUM_ET)
    def _(e):
        base_row = w * ROWS_PER_SUB

        @pl.loop(0, ROWS_PER_SUB // CH)
        def _(t):
            pltpu.async_copy(zrows_v, acc_sh.at[pl.ds(base_row + t * CH, CH)],
                             zsem)
            if with_counts:
                pltpu.async_copy(zcnt_v,
                                 cnt_sh.at[pl.ds(base_row + t * CH, CH)], zsem)

        pltpu.sync_copy(src_hbm.at[c, e, w], src_v)
        pltpu.sync_copy(dst_hbm.at[e, w], dst_v)

        @pl.loop(0, ROWS_PER_SUB // CH)
        def _(t):
            pltpu.make_async_copy(
                zrows_v, acc_sh.at[pl.ds(base_row + t * CH, CH)], zsem).wait()
            if with_counts:
                pltpu.make_async_copy(
                    zcnt_v, cnt_sh.at[pl.ds(base_row + t * CH, CH)],
                    zsem).wait()

        plsc.subcore_barrier()

        start_gather(0, 0)
        start_gather(1, 1)

        @pl.loop(0, (CPS - 1) // 3)
        def _(rr):
            for b in range(3):
                k = 3 * rr + b
                wait_gather(k, b)
                start_scatter(e, k, b)
                nb = (b + 2) % 3

                @pl.when(k >= 1)
                def _():
                    wait_scatter(e, k - 1, nb)

                @pl.when(k + 2 <= CPS - 1)
                def _():
                    start_gather(k + 2, nb)

        wait_gather(CPS - 1, (CPS - 1) % 3)
        start_scatter(e, CPS - 1, (CPS - 1) % 3)
        wait_scatter(e, CPS - 2, (CPS - 2) % 3)
        wait_scatter(e, CPS - 1, (CPS - 1) % 3)

        plsc.subcore_barrier()
        pltpu.sync_copy(acc_sh.at[pl.ds(base_row, ROWS_PER_SUB)],
                        sums_hbm.at[c, e, pl.ds(base_row, ROWS_PER_SUB)])
        if with_counts:
            @pl.when(c == e // 5)
            def _():
                pltpu.sync_copy(cnt_sh.at[pl.ds(base_row, ROWS_PER_SUB)],
                                cnts_hbm.at[e, pl.ds(base_row, ROWS_PER_SUB)])


def _sc_segsum(wh_flat, src_g2, dst_l, with_counts):
    outs = [jax.ShapeDtypeStruct((2, NUM_ET, N_PAD, HD), jnp.float32)]
    scratch = [
        pltpu.VMEM((CPS, CH), jnp.int32),
        pltpu.VMEM((CPS, CH), jnp.int32),
        pltpu.VMEM((CH, HD), jnp.float32),
        pltpu.VMEM((CH, HD), jnp.float32),
        pltpu.VMEM((CH, HD), jnp.float32),
    ]
    if with_counts:
        outs.append(jax.ShapeDtypeStruct((NUM_ET, N_PAD, 16), jnp.float32))
        scratch.append(pltpu.VMEM((CH, 16), jnp.float32))
    scratch.append(pltpu.VMEM((CH, HD), jnp.float32))
    if with_counts:
        scratch.append(pltpu.VMEM((CH, 16), jnp.float32))
    scratch.append(pltpu.VMEM_SHARED((N_PAD, HD), jnp.float32))
    if with_counts:
        scratch.append(pltpu.VMEM_SHARED((N_PAD, 16), jnp.float32))
    for _ in range(7):
        scratch.append(pltpu.SemaphoreType.DMA)

    k = pl.kernel(
        functools.partial(_segsum_body, with_counts),
        out_type=tuple(outs),
        mesh=_mesh,
        scratch_types=scratch,
        compiler_params=pltpu.CompilerParams(use_tc_tiling_on_sc=False),
    )
    return k(wh_flat, src_g2, dst_l)


_G0 = [e for e in range(NUM_ET) if _DST[e] == 0]
_G1 = [e for e in range(NUM_ET) if _DST[e] == 1]
_G2 = [e for e in range(NUM_ET) if _DST[e] == 2]


def _mean_body(leaky, emit_bf16, s_ref, c_ref, o_ref, *o2_ref):
    cnt = jnp.maximum(c_ref[:, :, 0:1], 1.0)
    halves = []
    for h in range(2):
        m = s_ref[h] / cnt
        hs = []
        for grp in (_G0, _G1, _G2):
            acc = m[grp[0]]
            for e in grp[1:]:
                acc = acc + m[e]
            hs.append(acc)
        halves.append(jnp.stack(hs, axis=0))
    out = jnp.concatenate(halves, axis=-1)
    if leaky:
        out = jnp.where(out >= 0.0, out, 0.01 * out)
    o_ref[...] = out
    if emit_bf16:
        o2_ref[0][...] = out.astype(jnp.bfloat16)


def _mean_stage(sums, cnts, leaky, emit_bf16=False):
    bn = 512
    out_shapes = [jax.ShapeDtypeStruct((3, N_PAD, D), jnp.float32)]
    out_specs = [pl.BlockSpec((3, bn, D), lambda i: (0, i, 0))]
    if emit_bf16:
        out_shapes.append(jax.ShapeDtypeStruct((3, N_PAD, D), jnp.bfloat16))
        out_specs.append(pl.BlockSpec((3, bn, D), lambda i: (0, i, 0)))
    res = pl.pallas_call(
        functools.partial(_mean_body, leaky, emit_bf16),
        grid=(N_PAD // bn,),
        in_specs=[
            pl.BlockSpec((2, NUM_ET, bn, HD), lambda i: (0, 0, i, 0)),
            pl.BlockSpec((NUM_ET, bn, 16), lambda i: (0, i, 0)),
        ],
        out_specs=out_specs,
        out_shape=out_shapes,
    )(sums, cnts)
    return res if emit_bf16 else res[0]


def _gtable_body(h_ref, w_ref, g_ref):
    h = h_ref[...]
    w = w_ref[...]
    g_ref[...] = jnp.stack(
        [h[_DST16[s]] * w[_ET16[s]][None, :] for s in range(16)],
        axis=0).astype(jnp.bfloat16)


def _g_table(h2, w_rels):
    bn = 512
    return pl.pallas_call(
        _gtable_body,
        grid=(N_PAD // bn,),
        in_specs=[
            pl.BlockSpec((3, bn, D), lambda i: (0, i, 0)),
            pl.BlockSpec((NUM_ET, D), lambda i: (0, 0)),
        ],
        out_specs=pl.BlockSpec((16, bn, D), lambda i: (0, i, 0)),
        out_shape=jax.ShapeDtypeStruct((16, N_PAD, D), jnp.bfloat16),
    )(h2, w_rels)


def _distmult_body(h_hbm, g_hbm, src_hbm, dst_hbm, p_hbm,
                   src_v, dst_v, u0_v, u1_v, g0_v, g1_v, p_v,
                   semu0, semu1, semg0, semg1):
    c = lax.axis_index("c")
    w = lax.axis_index("s")
    us = (u0_v, u1_v)
    gs = (g0_v, g1_v)
    semus = (semu0, semu1)
    semgs = (semg0, semg1)

    def start_gathers(k, b):
        pltpu.async_copy(h_hbm.at[src_v.at[k]], us[b], semus[b])
        pltpu.async_copy(g_hbm.at[dst_v.at[k]], gs[b], semgs[b])

    def compute_chunk(k, b):
        pltpu.make_async_copy(h_hbm.at[src_v.at[k]], us[b], semus[b]).wait()
        pltpu.make_async_copy(g_hbm.at[dst_v.at[k]], gs[b], semgs[b]).wait()
        u_v, g_v = us[b], gs[b]

        @pl.loop(0, CH // 8)
        def _(r8):
            for i in range(8):
                r = r8 * 8 + i
                acc = None
                for q in range(4):
                    uq = plsc.bitcast(u_v[r, pl.ds(16 * q, 16)], jnp.bfloat16)
                    gq = plsc.bitcast(g_v[r, pl.ds(16 * q, 16)], jnp.bfloat16)
                    ua, ub = plsc.unpack(
                        uq, format=plsc.PackFormat.INTERLEAVED)
                    ga, gb = plsc.unpack(
                        gq, format=plsc.PackFormat.INTERLEAVED)
                    t = ua * ga + ub * gb
                    acc = t if acc is None else acc + t
                p_v[k, r8, pl.ds(16 * i, 16)] = acc

    @pl.loop(0, 8)
    def _(j):
        s = c * 8 + j
        pltpu.sync_copy(src_hbm.at[s, w], src_v)
        pltpu.sync_copy(dst_hbm.at[s, w], dst_v)

        start_gathers(0, 0)

        @pl.loop(0, (CPS - 1) // 2)
        def _(kk):
            k = 2 * kk
            start_gathers(k + 1, 1)
            compute_chunk(k, 0)
            start_gathers(k + 2, 0)
            compute_chunk(k + 1, 1)

        compute_chunk(CPS - 1, 0)

        pltpu.sync_copy(p_v, p_hbm.at[s, w])


def _sc_distmult(h2_flat, g_flat, src16, dst16):
    k = pl.kernel(
        _distmult_body,
        out_type=jax.ShapeDtypeStruct((16, NSUB, CPS, CH // 8, 128),
                                      jnp.float32),
        mesh=_mesh,
        scratch_types=[
            pltpu.VMEM((CPS, CH), jnp.int32),
            pltpu.VMEM((CPS, CH), jnp.int32),
            pltpu.VMEM((CH, HD), jnp.int32),
            pltpu.VMEM((CH, HD), jnp.int32),
            pltpu.VMEM((CH, HD), jnp.int32),
            pltpu.VMEM((CH, HD), jnp.int32),
            pltpu.VMEM((CPS, CH // 8, 128), jnp.float32),
            pltpu.SemaphoreType.DMA,
            pltpu.SemaphoreType.DMA,
            pltpu.SemaphoreType.DMA,
            pltpu.SemaphoreType.DMA,
        ],
        compiler_params=pltpu.CompilerParams(needs_layout_passes=False,
                                             use_tc_tiling_on_sc=False),
    )
    return k(h2_flat, g_flat, src16, dst16)


def _finish_body(p_ref, s_ref, o_ref):
    o_ref[0] = jnp.dot(p_ref[0], s_ref[...],
                       preferred_element_type=jnp.float32)


def _finish(p16):
    sel = np.zeros((128, 8), np.float32)
    for d in range(128):
        sel[d, d // 16] = 1.0
    sel = jnp.asarray(sel)
    bn = 3200
    return pl.pallas_call(
        _finish_body,
        grid=(16, (E_PAD // 8) // bn),
        in_specs=[
            pl.BlockSpec((1, bn, 128), lambda s, i: (s, i, 0)),
            pl.BlockSpec((128, 8), lambda s, i: (0, 0)),
        ],
        out_specs=pl.BlockSpec((1, bn, 8), lambda s, i: (s, i, 0)),
        out_shape=jax.ShapeDtypeStruct((16, E_PAD // 8, 8), jnp.float32),
    )(p16, sel)


def kernel(x_disease, x_drug, x_gene, W1, W2, w_rels, edges_all, neg_edges):
    x3 = jnp.stack([x_disease, x_drug, x_gene], axis=0)
    x3 = jnp.pad(x3, ((0, 0), (0, N_PAD - N), (0, 0)))

    src_l = jnp.pad(edges_all[:, 0, :], ((0, 0), (0, E_PAD - E)))
    dst_l = jnp.pad(edges_all[:, 1, :], ((0, 0), (0, E_PAD - E)),
                    constant_values=N)
    et_off = (jnp.arange(NUM_ET, dtype=jnp.int32) * N_PAD)[:, None]
    src_g = src_l + et_off
    src_g2 = jnp.stack([src_g, src_g + NUM_ET * N_PAD], axis=0
                       ).reshape(2, NUM_ET, NSUB, CPS, CH)
    dst_l = dst_l.reshape(NUM_ET, NSUB, CPS, CH)

    s16 = jnp.concatenate([edges_all[:, 0, :], neg_edges[:, 0, :]], axis=0)
    d16 = jnp.concatenate([edges_all[:, 1, :], neg_edges[:, 1, :]], axis=0)
    s16 = jnp.pad(s16, ((0, 0), (0, E_PAD - E)))
    d16 = jnp.pad(d16, ((0, 0), (0, E_PAD - E)))
    src16 = (s16 + (jnp.asarray(_SRC16, jnp.int32) * N_PAD)[:, None]
             ).reshape(16, NSUB, CPS, CH)
    dst16 = (d16 + (jnp.arange(16, dtype=jnp.int32) * N_PAD)[:, None]
             ).reshape(16, NSUB, CPS, CH)

    wh1 = _per_etype_matmul(x3, W1[:, :, :HD], W1[:, :, HD:]
                            ).reshape(2 * NUM_ET * N_PAD, HD)
    sums1, cnts = _sc_segsum(wh1, src_g2, dst_l, with_counts=True)
    h1 = _mean_stage(sums1, cnts, leaky=True)

    wh2 = _per_etype_matmul(h1, W2[:, :, :HD], W2[:, :, HD:]
                            ).reshape(2 * NUM_ET * N_PAD, HD)
    (sums2,) = _sc_segsum(wh2, src_g2, dst_l, with_counts=False)
    h2, h2b = _mean_stage(sums2, cnts, leaky=False, emit_bf16=True)

    g16 = lax.bitcast_convert_type(
        _g_table(h2, w_rels).reshape(16 * N_PAD, HD, 2), jnp.int32)
    h2_flat = lax.bitcast_convert_type(
        h2b.reshape(3 * N_PAD, HD, 2), jnp.int32)
    p16 = _sc_distmult(h2_flat, g16, src16, dst16)
    scores = _finish(p16.reshape(16, E_PAD // 8, 128))
    return scores.reshape(16, E_PAD)[:, :E].reshape(-1)


# speedup vs baseline: 1.0045x; 1.0045x over previous
import functools

import jax
import jax.numpy as jnp
import numpy as np
from jax import lax
from jax.experimental import pallas as pl
from jax.experimental.pallas import tpu as pltpu
from jax.experimental.pallas import tpu_sc as plsc

N = 10000
D = 128
HD = D // 2
E = 50000
NUM_ET = 10

_SRC = [1, 1, 1, 0, 0, 0, 2, 0, 1, 2]
_DST = [0, 0, 0, 1, 1, 1, 2, 0, 2, 0]

_ET16 = list(range(10)) + list(range(6))
_SRC16 = [_SRC[e] for e in _ET16]
_DST16 = [_DST[e] for e in _ET16]

N_PAD = 10240
E_PAD = 51200
CH = 128
CHUNKS = E_PAD // CH
NSUB = 16
NCORE = 2
CPS = CHUNKS // NSUB
ROWS_PER_SUB = N_PAD // NSUB

_mesh = plsc.VectorSubcoreMesh(core_axis_name="c", subcore_axis_name="s",
                               num_cores=NCORE, num_subcores=NSUB)


def _mm_body(x_ref, wl_ref, wr_ref, o_ref):
    x = x_ref[0]
    o_ref[0, 0] = jnp.dot(x, wl_ref[0], preferred_element_type=jnp.float32)
    o_ref[1, 0] = jnp.dot(x, wr_ref[0], preferred_element_type=jnp.float32)


def _per_etype_matmul(x3, wl, wr):
    bn = 512
    return pl.pallas_call(
        _mm_body,
        grid=(NUM_ET, N_PAD // bn),
        in_specs=[
            pl.BlockSpec((1, bn, D),
                         lambda e, i: (jnp.where(
                             (e < 3) | (e == 8), 1,
                             jnp.where((e == 6) | (e == 9), 2, 0)), i, 0)),
            pl.BlockSpec((1, D, HD), lambda e, i: (e, 0, 0)),
            pl.BlockSpec((1, D, HD), lambda e, i: (e, 0, 0)),
        ],
        out_specs=pl.BlockSpec((2, 1, bn, HD), lambda e, i: (0, e, i, 0)),
        out_shape=jax.ShapeDtypeStruct((2, NUM_ET, N_PAD, HD), jnp.float32),
    )(x3, wl, wr)


def _segsum_body(with_counts, wh_hbm, src_hbm, dst_hbm, *refs):
    if with_counts:
        (sums_hbm, cnts_hbm, src_v, dst_v, rows0_v, rows1_v, rows2_v, ones_v,
         zrows_v, zcnt_v, acc_sh, cnt_sh, g0, g1, g2, s0, s1, s2, zsem) = refs
    else:
        (sums_hbm, src_v, dst_v, rows0_v, rows1_v, rows2_v, zrows_v, acc_sh,
         g0, g1, g2, s0, s1, s2, zsem) = refs
    rows = (rows0_v, rows1_v, rows2_v)
    sems = (g0, g1, g2)
    ssems = (s0, s1, s2)
    c = lax.axis_index("c")
    w = lax.axis_index("s")
    zero16 = jnp.zeros((16,), jnp.float32)
    one16 = jnp.ones((16,), jnp.float32)

    @pl.loop(0, CH)
    def _(r):
        for k in range(HD // 16):
            zrows_v[r, pl.ds(16 * k, 16)] = zero16
        if with_counts:
            zcnt_v[r, pl.ds(0, 16)] = zero16
            ones_v[r, pl.ds(0, 16)] = one16

    def start_gather(k, b):
        pltpu.async_copy(wh_hbm.at[src_v.at[k]], rows[b], sems[b])

    def wait_gather(k, b):
        pltpu.make_async_copy(wh_hbm.at[src_v.at[k]], rows[b], sems[b]).wait()

    def start_scatter(e, k, b):
        pltpu.async_copy(rows[b], acc_sh.at[dst_v.at[k]], ssems[b], add=True)
        if with_counts:
            @pl.when(c == e // 5)
            def _():
                pltpu.async_copy(ones_v, cnt_sh.at[dst_v.at[k]], ssems[b],
                                 add=True)

    def wait_scatter(e, k, b):
        pltpu.make_async_copy(rows[b], acc_sh.at[dst_v.at[k]],
                              ssems[b]).wait()
        if with_counts:
            @pl.when(c == e // 5)
            def _():
                pltpu.make_async_copy(ones_v, cnt_sh.at[dst_v.at[k]],
                                      ssems[b]).wait()

    @pl.loop(0, NUM_ET)
    def _(e):
        base_row = w * ROWS_PER_SUB

        @pl.loop(0, ROWS_PER_SUB // CH)
        def _(t):
            pltpu.async_copy(zrows_v, acc_sh.at[pl.ds(base_row + t * CH, CH)],
                             zsem)
            if with_counts:
                pltpu.async_copy(zcnt_v,
                                 cnt_sh.at[pl.ds(base_row + t * CH, CH)], zsem)

        pltpu.sync_copy(src_hbm.at[c, e, w], src_v)
        pltpu.sync_copy(dst_hbm.at[e, w], dst_v)

        @pl.loop(0, ROWS_PER_SUB // CH)
        def _(t):
            pltpu.make_async_copy(
                zrows_v, acc_sh.at[pl.ds(base_row + t * CH, CH)], zsem).wait()
            if with_counts:
                pltpu.make_async_copy(
                    zcnt_v, cnt_sh.at[pl.ds(base_row + t * CH, CH)],
                    zsem).wait()

        plsc.subcore_barrier()

        start_gather(0, 0)
        start_gather(1, 1)

        @pl.loop(0, (CPS - 1) // 3)
        def _(rr):
            for b in range(3):
                k = 3 * rr + b
                wait_gather(k, b)
                start_scatter(e, k, b)
                nb = (b + 2) % 3

                @pl.when(k >= 1)
                def _():
                    wait_scatter(e, k - 1, nb)

                @pl.when(k + 2 <= CPS - 1)
                def _():
                    start_gather(k + 2, nb)

        wait_gather(CPS - 1, (CPS - 1) % 3)
        start_scatter(e, CPS - 1, (CPS - 1) % 3)
        wait_scatter(e, CPS - 2, (CPS - 2) % 3)
        wait_scatter(e, CPS - 1, (CPS - 1) % 3)

        plsc.subcore_barrier()
        pltpu.sync_copy(acc_sh.at[pl.ds(base_row, ROWS_PER_SUB)],
                        sums_hbm.at[c, e, pl.ds(base_row, ROWS_PER_SUB)])
        if with_counts:
            @pl.when(c == e // 5)
            def _():
                pltpu.sync_copy(cnt_sh.at[pl.ds(base_row, ROWS_PER_SUB)],
                                cnts_hbm.at[e, pl.ds(base_row, ROWS_PER_SUB)])


def _sc_segsum(wh_flat, src_g2, dst_l, with_counts):
    outs = [jax.ShapeDtypeStruct((2, NUM_ET, N_PAD, HD), jnp.float32)]
    scratch = [
        pltpu.VMEM((CPS, CH), jnp.int32),
        pltpu.VMEM((CPS, CH), jnp.int32),
        pltpu.VMEM((CH, HD), jnp.float32),
        pltpu.VMEM((CH, HD), jnp.float32),
        pltpu.VMEM((CH, HD), jnp.float32),
    ]
    if with_counts:
        outs.append(jax.ShapeDtypeStruct((NUM_ET, N_PAD, 16), jnp.float32))
        scratch.append(pltpu.VMEM((CH, 16), jnp.float32))
    scratch.append(pltpu.VMEM((CH, HD), jnp.float32))
    if with_counts:
        scratch.append(pltpu.VMEM((CH, 16), jnp.float32))
    scratch.append(pltpu.VMEM_SHARED((N_PAD, HD), jnp.float32))
    if with_counts:
        scratch.append(pltpu.VMEM_SHARED((N_PAD, 16), jnp.float32))
    for _ in range(7):
        scratch.append(pltpu.SemaphoreType.DMA)

    k = pl.kernel(
        functools.partial(_segsum_body, with_counts),
        out_type=tuple(outs),
        mesh=_mesh,
        scratch_types=scratch,
        compiler_params=pltpu.CompilerParams(use_tc_tiling_on_sc=False),
    )
    return k(wh_flat, src_g2, dst_l)


_G0 = [e for e in range(NUM_ET) if _DST[e] == 0]
_G1 = [e for e in range(NUM_ET) if _DST[e] == 1]
_G2 = [e for e in range(NUM_ET) if _DST[e] == 2]


def _mean_body(leaky, emit_bf16, s_ref, c_ref, o_ref, *o2_ref):
    cnt = jnp.maximum(c_ref[:, :, 0:1], 1.0)
    halves = []
    for h in range(2):
        m = s_ref[h] / cnt
        hs = []
        for grp in (_G0, _G1, _G2):
            acc = m[grp[0]]
            for e in grp[1:]:
                acc = acc + m[e]
            hs.append(acc)
        halves.append(jnp.stack(hs, axis=0))
    out = jnp.concatenate(halves, axis=-1)
    if leaky:
        out = jnp.where(out >= 0.0, out, 0.01 * out)
    o_ref[...] = out
    if emit_bf16:
        o2_ref[0][...] = out.astype(jnp.bfloat16)


def _mean_stage(sums, cnts, leaky, emit_bf16=False):
    bn = 512
    out_shapes = [jax.ShapeDtypeStruct((3, N_PAD, D), jnp.float32)]
    out_specs = [pl.BlockSpec((3, bn, D), lambda i: (0, i, 0))]
    if emit_bf16:
        out_shapes.append(jax.ShapeDtypeStruct((3, N_PAD, D), jnp.bfloat16))
        out_specs.append(pl.BlockSpec((3, bn, D), lambda i: (0, i, 0)))
    res = pl.pallas_call(
        functools.partial(_mean_body, leaky, emit_bf16),
        grid=(N_PAD // bn,),
        in_specs=[
            pl.BlockSpec((2, NUM_ET, bn, HD), lambda i: (0, 0, i, 0)),
            pl.BlockSpec((NUM_ET, bn, 16), lambda i: (0, i, 0)),
        ],
        out_specs=out_specs,
        out_shape=out_shapes,
    )(sums, cnts)
    return res if emit_bf16 else res[0]


def _gtable_body(h_ref, w_ref, g_ref):
    h = h_ref[...]
    w = w_ref[...]
    g_ref[...] = jnp.stack(
        [h[_DST16[s]] * w[_ET16[s]][None, :] for s in range(16)],
        axis=0).astype(jnp.bfloat16)


def _g_table(h2, w_rels):
    bn = 512
    return pl.pallas_call(
        _gtable_body,
        grid=(N_PAD // bn,),
        in_specs=[
            pl.BlockSpec((3, bn, D), lambda i: (0, i, 0)),
            pl.BlockSpec((NUM_ET, D), lambda i: (0, 0)),
        ],
        out_specs=pl.BlockSpec((16, bn, D), lambda i: (0, i, 0)),
        out_shape=jax.ShapeDtypeStruct((16, N_PAD, D), jnp.bfloat16),
    )(h2, w_rels)


def _distmult_body(h_hbm, g_hbm, src_hbm, dst_hbm, p_hbm,
                   src_v, dst_v, u0_v, u1_v, g0_v, g1_v, p_v,
                   semu0, semu1, semg0, semg1):
    c = lax.axis_index("c")
    w = lax.axis_index("s")
    us = (u0_v, u1_v)
    gs = (g0_v, g1_v)
    semus = (semu0, semu1)
    semgs = (semg0, semg1)

    def start_gathers(k, b):
        pltpu.async_copy(h_hbm.at[src_v.at[k]], us[b], semus[b])
        pltpu.async_copy(g_hbm.at[dst_v.at[k]], gs[b], semgs[b])

    def compute_chunk(k, b):
        pltpu.make_async_copy(h_hbm.at[src_v.at[k]], us[b], semus[b]).wait()
        pltpu.make_async_copy(g_hbm.at[dst_v.at[k]], gs[b], semgs[b]).wait()
        u_v, g_v = us[b], gs[b]

        @pl.loop(0, CH // 8)
        def _(r8):
            for i in range(8):
                r = r8 * 8 + i
                acc = None
                for q in range(4):
                    uq = plsc.bitcast(u_v[r, pl.ds(16 * q, 16)], jnp.bfloat16)
                    gq = plsc.bitcast(g_v[r, pl.ds(16 * q, 16)], jnp.bfloat16)
                    t = uq * gq
                    acc = t if acc is None else acc + t
                pa, pb = plsc.unpack(acc, format=plsc.PackFormat.INTERLEAVED)
                p_v[k, r8, pl.ds(16 * i, 16)] = pa + pb

    @pl.loop(0, 8)
    def _(j):
        s = c * 8 + j
        pltpu.sync_copy(src_hbm.at[s, w], src_v)
        pltpu.sync_copy(dst_hbm.at[s, w], dst_v)

        start_gathers(0, 0)

        @pl.loop(0, (CPS - 1) // 2)
        def _(kk):
            k = 2 * kk
            start_gathers(k + 1, 1)
            compute_chunk(k, 0)
            start_gathers(k + 2, 0)
            compute_chunk(k + 1, 1)

        compute_chunk(CPS - 1, 0)

        pltpu.sync_copy(p_v, p_hbm.at[s, w])


def _sc_distmult(h2_flat, g_flat, src16, dst16):
    k = pl.kernel(
        _distmult_body,
        out_type=jax.ShapeDtypeStruct((16, NSUB, CPS, CH // 8, 128),
                                      jnp.float32),
        mesh=_mesh,
        scratch_types=[
            pltpu.VMEM((CPS, CH), jnp.int32),
            pltpu.VMEM((CPS, CH), jnp.int32),
            pltpu.VMEM((CH, HD), jnp.int32),
            pltpu.VMEM((CH, HD), jnp.int32),
            pltpu.VMEM((CH, HD), jnp.int32),
            pltpu.VMEM((CH, HD), jnp.int32),
            pltpu.VMEM((CPS, CH // 8, 128), jnp.float32),
            pltpu.SemaphoreType.DMA,
            pltpu.SemaphoreType.DMA,
            pltpu.SemaphoreType.DMA,
            pltpu.SemaphoreType.DMA,
        ],
        compiler_params=pltpu.CompilerParams(needs_layout_passes=False,
                                             use_tc_tiling_on_sc=False),
    )
    return k(h2_flat, g_flat, src16, dst16)


def _finish_body(p_ref, s_ref, o_ref):
    o_ref[0] = jnp.dot(p_ref[0], s_ref[...],
                       preferred_element_type=jnp.float32)


def _finish(p16):
    sel = np.zeros((128, 8), np.float32)
    for d in range(128):
        sel[d, d // 16] = 1.0
    sel = jnp.asarray(sel)
    bn = 3200
    return pl.pallas_call(
        _finish_body,
        grid=(16, (E_PAD // 8) // bn),
        in_specs=[
            pl.BlockSpec((1, bn, 128), lambda s, i: (s, i, 0)),
            pl.BlockSpec((128, 8), lambda s, i: (0, 0)),
        ],
        out_specs=pl.BlockSpec((1, bn, 8), lambda s, i: (s, i, 0)),
        out_shape=jax.ShapeDtypeStruct((16, E_PAD // 8, 8), jnp.float32),
    )(p16, sel)


def kernel(x_disease, x_drug, x_gene, W1, W2, w_rels, edges_all, neg_edges):
    x3 = jnp.stack([x_disease, x_drug, x_gene], axis=0)
    x3 = jnp.pad(x3, ((0, 0), (0, N_PAD - N), (0, 0)))

    src_l = jnp.pad(edges_all[:, 0, :], ((0, 0), (0, E_PAD - E)))
    dst_l = jnp.pad(edges_all[:, 1, :], ((0, 0), (0, E_PAD - E)),
                    constant_values=N)
    et_off = (jnp.arange(NUM_ET, dtype=jnp.int32) * N_PAD)[:, None]
    src_g = src_l + et_off
    src_g2 = jnp.stack([src_g, src_g + NUM_ET * N_PAD], axis=0
                       ).reshape(2, NUM_ET, NSUB, CPS, CH)
    dst_l = dst_l.reshape(NUM_ET, NSUB, CPS, CH)

    s16 = jnp.concatenate([edges_all[:, 0, :], neg_edges[:, 0, :]], axis=0)
    d16 = jnp.concatenate([edges_all[:, 1, :], neg_edges[:, 1, :]], axis=0)
    s16 = jnp.pad(s16, ((0, 0), (0, E_PAD - E)))
    d16 = jnp.pad(d16, ((0, 0), (0, E_PAD - E)))
    src16 = (s16 + (jnp.asarray(_SRC16, jnp.int32) * N_PAD)[:, None]
             ).reshape(16, NSUB, CPS, CH)
    dst16 = (d16 + (jnp.arange(16, dtype=jnp.int32) * N_PAD)[:, None]
             ).reshape(16, NSUB, CPS, CH)

    wh1 = _per_etype_matmul(x3, W1[:, :, :HD], W1[:, :, HD:]
                            ).reshape(2 * NUM_ET * N_PAD, HD)
    sums1, cnts = _sc_segsum(wh1, src_g2, dst_l, with_counts=True)
    h1 = _mean_stage(sums1, cnts, leaky=True)

    wh2 = _per_etype_matmul(h1, W2[:, :, :HD], W2[:, :, HD:]
                            ).reshape(2 * NUM_ET * N_PAD, HD)
    (sums2,) = _sc_segsum(wh2, src_g2, dst_l, with_counts=False)
    h2, h2b = _mean_stage(sums2, cnts, leaky=False, emit_bf16=True)

    g16 = lax.bitcast_convert_type(
        _g_table(h2, w_rels).reshape(16 * N_PAD, HD, 2), jnp.int32)
    h2_flat = lax.bitcast_convert_type(
        h2b.reshape(3 * N_PAD, HD, 2), jnp.int32)
    p16 = _sc_distmult(h2_flat, g16, src16, dst16)
    scores = _finish(p16.reshape(16, E_PAD // 8, 128))
    return scores.reshape(16, E_PAD)[:, :E].reshape(-1)


# speedup vs baseline: 1.1956x; 1.1902x over previous
import functools

import jax
import jax.numpy as jnp
import numpy as np
from jax import lax
from jax.experimental import pallas as pl
from jax.experimental.pallas import tpu as pltpu
from jax.experimental.pallas import tpu_sc as plsc

N = 10000
D = 128
HD = D // 2
E = 50000
NUM_ET = 10

_SRC = [1, 1, 1, 0, 0, 0, 2, 0, 1, 2]
_DST = [0, 0, 0, 1, 1, 1, 2, 0, 2, 0]

_ET16 = list(range(10)) + list(range(6))
_SRC16 = [_SRC[e] for e in _ET16]
_DST16 = [_DST[e] for e in _ET16]

N_PAD = 10240
E_PAD = 51200
CH = 128
CHUNKS = E_PAD // CH
NSUB = 16
NCORE = 2
CPS = CHUNKS // NSUB
ROWS_PER_SUB = N_PAD // NSUB

_mesh = plsc.VectorSubcoreMesh(core_axis_name="c", subcore_axis_name="s",
                               num_cores=NCORE, num_subcores=NSUB)


def _mm_body(x_ref, wl_ref, wr_ref, o_ref):
    x = x_ref[0]
    o_ref[0, 0] = jnp.dot(x, wl_ref[0], preferred_element_type=jnp.float32)
    o_ref[1, 0] = jnp.dot(x, wr_ref[0], preferred_element_type=jnp.float32)


def _per_etype_matmul(x3, wl, wr):
    bn = 512
    return pl.pallas_call(
        _mm_body,
        grid=(NUM_ET, N_PAD // bn),
        in_specs=[
            pl.BlockSpec((1, bn, D),
                         lambda e, i: (jnp.where(
                             (e < 3) | (e == 8), 1,
                             jnp.where((e == 6) | (e == 9), 2, 0)), i, 0)),
            pl.BlockSpec((1, D, HD), lambda e, i: (e, 0, 0)),
            pl.BlockSpec((1, D, HD), lambda e, i: (e, 0, 0)),
        ],
        out_specs=pl.BlockSpec((2, 1, bn, HD), lambda e, i: (0, e, i, 0)),
        out_shape=jax.ShapeDtypeStruct((2, NUM_ET, N_PAD, HD), jnp.float32),
    )(x3, wl, wr)


def _segsum_body(with_counts, wh_hbm, src_hbm, dst_hbm, *refs):
    if with_counts:
        (sums_hbm, cnts_hbm, src_v, dst_v, rows0_v, rows1_v, rows2_v, ones_v,
         zrows_v, zcnt_v, acc_sh, cnt_sh, g0, g1, g2, s0, s1, s2, zsem) = refs
    else:
        (sums_hbm, src_v, dst_v, rows0_v, rows1_v, rows2_v, zrows_v, acc_sh,
         g0, g1, g2, s0, s1, s2, zsem) = refs
    rows = (rows0_v, rows1_v, rows2_v)
    sems = (g0, g1, g2)
    ssems = (s0, s1, s2)
    c = lax.axis_index("c")
    w = lax.axis_index("s")
    zero16 = jnp.zeros((16,), jnp.float32)
    one16 = jnp.ones((16,), jnp.float32)

    @pl.loop(0, CH)
    def _(r):
        for k in range(HD // 16):
            zrows_v[r, pl.ds(16 * k, 16)] = zero16
        if with_counts:
            zcnt_v[r, pl.ds(0, 16)] = zero16
            ones_v[r, pl.ds(0, 16)] = one16

    def start_gather(k, b):
        pltpu.async_copy(wh_hbm.at[src_v.at[k]], rows[b], sems[b])

    def wait_gather(k, b):
        pltpu.make_async_copy(wh_hbm.at[src_v.at[k]], rows[b], sems[b]).wait()

    def start_scatter(e, k, b):
        pltpu.async_copy(rows[b], acc_sh.at[dst_v.at[k]], ssems[b], add=True)
        if with_counts:
            @pl.when(c == e // 5)
            def _():
                pltpu.async_copy(ones_v, cnt_sh.at[dst_v.at[k]], ssems[b],
                                 add=True)

    def wait_scatter(e, k, b):
        pltpu.make_async_copy(rows[b], acc_sh.at[dst_v.at[k]],
                              ssems[b]).wait()
        if with_counts:
            @pl.when(c == e // 5)
            def _():
                pltpu.make_async_copy(ones_v, cnt_sh.at[dst_v.at[k]],
                                      ssems[b]).wait()

    @pl.loop(0, NUM_ET)
    def _(e):
        base_row = w * ROWS_PER_SUB

        @pl.loop(0, ROWS_PER_SUB // CH)
        def _(t):
            pltpu.async_copy(zrows_v, acc_sh.at[pl.ds(base_row + t * CH, CH)],
                             zsem)
            if with_counts:
                pltpu.async_copy(zcnt_v,
                                 cnt_sh.at[pl.ds(base_row + t * CH, CH)], zsem)

        pltpu.sync_copy(src_hbm.at[c, e, w], src_v)
        pltpu.sync_copy(dst_hbm.at[e, w], dst_v)

        @pl.loop(0, ROWS_PER_SUB // CH)
        def _(t):
            pltpu.make_async_copy(
                zrows_v, acc_sh.at[pl.ds(base_row + t * CH, CH)], zsem).wait()
            if with_counts:
                pltpu.make_async_copy(
                    zcnt_v, cnt_sh.at[pl.ds(base_row + t * CH, CH)],
                    zsem).wait()

        plsc.subcore_barrier()

        start_gather(0, 0)
        start_gather(1, 1)

        @pl.loop(0, (CPS - 1) // 3)
        def _(rr):
            for b in range(3):
                k = 3 * rr + b
                wait_gather(k, b)
                start_scatter(e, k, b)
                nb = (b + 2) % 3

                @pl.when(k >= 1)
                def _():
                    wait_scatter(e, k - 1, nb)

                @pl.when(k + 2 <= CPS - 1)
                def _():
                    start_gather(k + 2, nb)

        wait_gather(CPS - 1, (CPS - 1) % 3)
        start_scatter(e, CPS - 1, (CPS - 1) % 3)
        wait_scatter(e, CPS - 2, (CPS - 2) % 3)
        wait_scatter(e, CPS - 1, (CPS - 1) % 3)

        plsc.subcore_barrier()
        pltpu.sync_copy(acc_sh.at[pl.ds(base_row, ROWS_PER_SUB)],
                        sums_hbm.at[c, e, pl.ds(base_row, ROWS_PER_SUB)])
        if with_counts:
            @pl.when(c == e // 5)
            def _():
                pltpu.sync_copy(cnt_sh.at[pl.ds(base_row, ROWS_PER_SUB)],
                                cnts_hbm.at[e, pl.ds(base_row, ROWS_PER_SUB)])


def _sc_segsum(wh_flat, src_g2, dst_l, with_counts):
    outs = [jax.ShapeDtypeStruct((2, NUM_ET, N_PAD, HD), jnp.float32)]
    scratch = [
        pltpu.VMEM((CPS, CH), jnp.int32),
        pltpu.VMEM((CPS, CH), jnp.int32),
        pltpu.VMEM((CH, HD), jnp.float32),
        pltpu.VMEM((CH, HD), jnp.float32),
        pltpu.VMEM((CH, HD), jnp.float32),
    ]
    if with_counts:
        outs.append(jax.ShapeDtypeStruct((NUM_ET, N_PAD, 16), jnp.float32))
        scratch.append(pltpu.VMEM((CH, 16), jnp.float32))
    scratch.append(pltpu.VMEM((CH, HD), jnp.float32))
    if with_counts:
        scratch.append(pltpu.VMEM((CH, 16), jnp.float32))
    scratch.append(pltpu.VMEM_SHARED((N_PAD, HD), jnp.float32))
    if with_counts:
        scratch.append(pltpu.VMEM_SHARED((N_PAD, 16), jnp.float32))
    for _ in range(7):
        scratch.append(pltpu.SemaphoreType.DMA)

    k = pl.kernel(
        functools.partial(_segsum_body, with_counts),
        out_type=tuple(outs),
        mesh=_mesh,
        scratch_types=scratch,
        compiler_params=pltpu.CompilerParams(use_tc_tiling_on_sc=False),
    )
    return k(wh_flat, src_g2, dst_l)


_G0 = [e for e in range(NUM_ET) if _DST[e] == 0]
_G1 = [e for e in range(NUM_ET) if _DST[e] == 1]
_G2 = [e for e in range(NUM_ET) if _DST[e] == 2]


def _mean_reduce(s_ref, c_ref):
    cnt = jnp.maximum(c_ref[:, :, 0:1], 1.0)
    halves = []
    for h in range(2):
        m = s_ref[h] / cnt
        hs = []
        for grp in (_G0, _G1, _G2):
            acc = m[grp[0]]
            for e in grp[1:]:
                acc = acc + m[e]
            hs.append(acc)
        halves.append(jnp.stack(hs, axis=0))
    return jnp.concatenate(halves, axis=-1)


def _mean_body(s_ref, c_ref, o_ref):
    out = _mean_reduce(s_ref, c_ref)
    o_ref[...] = jnp.where(out >= 0.0, out, 0.01 * out)


def _mean_stage(sums, cnts):
    bn = 512
    return pl.pallas_call(
        _mean_body,
        grid=(N_PAD // bn,),
        in_specs=[
            pl.BlockSpec((2, NUM_ET, bn, HD), lambda i: (0, 0, i, 0)),
            pl.BlockSpec((NUM_ET, bn, 16), lambda i: (0, i, 0)),
        ],
        out_specs=pl.BlockSpec((3, bn, D), lambda i: (0, i, 0)),
        out_shape=jax.ShapeDtypeStruct((3, N_PAD, D), jnp.float32),
    )(sums, cnts)


def _pack_bf16_pair(x):
    lo = lax.bitcast_convert_type(
        x[..., :HD].astype(jnp.bfloat16), jnp.uint16).astype(jnp.uint32)
    hi = lax.bitcast_convert_type(
        x[..., HD:].astype(jnp.bfloat16), jnp.uint16).astype(jnp.uint32)
    return lax.bitcast_convert_type(lo | (hi << 16), jnp.int32)


def _mean2_body(s_ref, c_ref, w_ref, h_ref, g_ref):
    out = _mean_reduce(s_ref, c_ref)
    w = w_ref[...]
    h_ref[...] = _pack_bf16_pair(out)
    g_ref[...] = _pack_bf16_pair(jnp.stack(
        [out[_DST16[s]] * w[_ET16[s]][None, :] for s in range(16)], axis=0))


def _mean2_pack(sums, cnts, w_rels):
    bn = 512
    return pl.pallas_call(
        _mean2_body,
        grid=(N_PAD // bn,),
        in_specs=[
            pl.BlockSpec((2, NUM_ET, bn, HD), lambda i: (0, 0, i, 0)),
            pl.BlockSpec((NUM_ET, bn, 16), lambda i: (0, i, 0)),
            pl.BlockSpec((NUM_ET, D), lambda i: (0, 0)),
        ],
        out_specs=[
            pl.BlockSpec((3, bn, HD), lambda i: (0, i, 0)),
            pl.BlockSpec((16, bn, HD), lambda i: (0, i, 0)),
        ],
        out_shape=[
            jax.ShapeDtypeStruct((3, N_PAD, HD), jnp.int32),
            jax.ShapeDtypeStruct((16, N_PAD, HD), jnp.int32),
        ],
    )(sums, cnts, w_rels)


def _distmult_body(h_hbm, g_hbm, src_hbm, dst_hbm, p_hbm,
                   src_v, dst_v, u0_v, u1_v, g0_v, g1_v, p_v,
                   semu0, semu1, semg0, semg1):
    c = lax.axis_index("c")
    w = lax.axis_index("s")
    us = (u0_v, u1_v)
    gs = (g0_v, g1_v)
    semus = (semu0, semu1)
    semgs = (semg0, semg1)

    def start_gathers(k, b):
        pltpu.async_copy(h_hbm.at[src_v.at[k]], us[b], semus[b])
        pltpu.async_copy(g_hbm.at[dst_v.at[k]], gs[b], semgs[b])

    def compute_chunk(k, b):
        pltpu.make_async_copy(h_hbm.at[src_v.at[k]], us[b], semus[b]).wait()
        pltpu.make_async_copy(g_hbm.at[dst_v.at[k]], gs[b], semgs[b]).wait()
        u_v, g_v = us[b], gs[b]

        @pl.loop(0, CH // 8)
        def _(r8):
            for i in range(8):
                r = r8 * 8 + i
                acc = None
                for q in range(4):
                    uq = plsc.bitcast(u_v[r, pl.ds(16 * q, 16)], jnp.bfloat16)
                    gq = plsc.bitcast(g_v[r, pl.ds(16 * q, 16)], jnp.bfloat16)
                    t = uq * gq
                    acc = t if acc is None else acc + t
                pa, pb = plsc.unpack(acc, format=plsc.PackFormat.INTERLEAVED)
                p_v[k, r8, pl.ds(16 * i, 16)] = pa + pb

    @pl.loop(0, 8)
    def _(j):
        s = c * 8 + j
        pltpu.sync_copy(src_hbm.at[s, w], src_v)
        pltpu.sync_copy(dst_hbm.at[s, w], dst_v)

        start_gathers(0, 0)

        @pl.loop(0, (CPS - 1) // 2)
        def _(kk):
            k = 2 * kk
            start_gathers(k + 1, 1)
            compute_chunk(k, 0)
            start_gathers(k + 2, 0)
            compute_chunk(k + 1, 1)

        compute_chunk(CPS - 1, 0)

        pltpu.sync_copy(p_v, p_hbm.at[s, w])


def _sc_distmult(h2_flat, g_flat, src16, dst16):
    k = pl.kernel(
        _distmult_body,
        out_type=jax.ShapeDtypeStruct((16, NSUB, CPS, CH // 8, 128),
                                      jnp.float32),
        mesh=_mesh,
        scratch_types=[
            pltpu.VMEM((CPS, CH), jnp.int32),
            pltpu.VMEM((CPS, CH), jnp.int32),
            pltpu.VMEM((CH, HD), jnp.int32),
            pltpu.VMEM((CH, HD), jnp.int32),
            pltpu.VMEM((CH, HD), jnp.int32),
            pltpu.VMEM((CH, HD), jnp.int32),
            pltpu.VMEM((CPS, CH // 8, 128), jnp.float32),
            pltpu.SemaphoreType.DMA,
            pltpu.SemaphoreType.DMA,
            pltpu.SemaphoreType.DMA,
            pltpu.SemaphoreType.DMA,
        ],
        compiler_params=pltpu.CompilerParams(needs_layout_passes=False,
                                             use_tc_tiling_on_sc=False),
    )
    return k(h2_flat, g_flat, src16, dst16)


def _finish_body(p_ref, s_ref, o_ref):
    o_ref[0] = jnp.dot(p_ref[0], s_ref[...],
                       preferred_element_type=jnp.float32)


def _finish(p16):
    sel = np.zeros((128, 8), np.float32)
    for d in range(128):
        sel[d, d // 16] = 1.0
    sel = jnp.asarray(sel)
    bn = 3200
    return pl.pallas_call(
        _finish_body,
        grid=(16, (E_PAD // 8) // bn),
        in_specs=[
            pl.BlockSpec((1, bn, 128), lambda s, i: (s, i, 0)),
            pl.BlockSpec((128, 8), lambda s, i: (0, 0)),
        ],
        out_specs=pl.BlockSpec((1, bn, 8), lambda s, i: (s, i, 0)),
        out_shape=jax.ShapeDtypeStruct((16, E_PAD // 8, 8), jnp.float32),
    )(p16, sel)


def kernel(x_disease, x_drug, x_gene, W1, W2, w_rels, edges_all, neg_edges):
    x3 = jnp.stack([x_disease, x_drug, x_gene], axis=0)
    x3 = jnp.pad(x3, ((0, 0), (0, N_PAD - N), (0, 0)))

    src_l = jnp.pad(edges_all[:, 0, :], ((0, 0), (0, E_PAD - E)))
    dst_l = jnp.pad(edges_all[:, 1, :], ((0, 0), (0, E_PAD - E)),
                    constant_values=N)
    et_off = (jnp.arange(NUM_ET, dtype=jnp.int32) * N_PAD)[:, None]
    src_g = src_l + et_off
    src_g2 = jnp.stack([src_g, src_g + NUM_ET * N_PAD], axis=0
                       ).reshape(2, NUM_ET, NSUB, CPS, CH)
    dst_l = dst_l.reshape(NUM_ET, NSUB, CPS, CH)

    s16 = jnp.concatenate([edges_all[:, 0, :], neg_edges[:, 0, :]], axis=0)
    d16 = jnp.concatenate([edges_all[:, 1, :], neg_edges[:, 1, :]], axis=0)
    s16 = jnp.pad(s16, ((0, 0), (0, E_PAD - E)))
    d16 = jnp.pad(d16, ((0, 0), (0, E_PAD - E)))
    src16 = (s16 + (jnp.asarray(_SRC16, jnp.int32) * N_PAD)[:, None]
             ).reshape(16, NSUB, CPS, CH)
    dst16 = (d16 + (jnp.arange(16, dtype=jnp.int32) * N_PAD)[:, None]
             ).reshape(16, NSUB, CPS, CH)

    wh1 = _per_etype_matmul(x3, W1[:, :, :HD], W1[:, :, HD:]
                            ).reshape(2 * NUM_ET * N_PAD, HD)
    sums1, cnts = _sc_segsum(wh1, src_g2, dst_l, with_counts=True)
    h1 = _mean_stage(sums1, cnts)

    wh2 = _per_etype_matmul(h1, W2[:, :, :HD], W2[:, :, HD:]
                            ).reshape(2 * NUM_ET * N_PAD, HD)
    (sums2,) = _sc_segsum(wh2, src_g2, dst_l, with_counts=False)

    h2p, g16p = _mean2_pack(sums2, cnts, w_rels)
    p16 = _sc_distmult(h2p.reshape(3 * N_PAD, HD),
                       g16p.reshape(16 * N_PAD, HD), src16, dst16)
    scores = _finish(p16.reshape(16, E_PAD // 8, 128))
    return scores.reshape(16, E_PAD)[:, :E].reshape(-1)


# speedup vs baseline: 1.2725x; 1.0643x over previous
import functools

import jax
import jax.numpy as jnp
import numpy as np
from jax import lax
from jax.experimental import pallas as pl
from jax.experimental.pallas import tpu as pltpu
from jax.experimental.pallas import tpu_sc as plsc

N = 10000
D = 128
HD = D // 2
E = 50000
NUM_ET = 10

_SRC = [1, 1, 1, 0, 0, 0, 2, 0, 1, 2]
_DST = [0, 0, 0, 1, 1, 1, 2, 0, 2, 0]

_ET16 = list(range(10)) + list(range(6))
_SRC16 = [_SRC[e] for e in _ET16]
_DST16 = [_DST[e] for e in _ET16]

N_PAD = 10240
E_PAD = 51200
CH = 128
CHUNKS = E_PAD // CH
NSUB = 16
NCORE = 2
CPS = CHUNKS // NSUB
ROWS_PER_SUB = N_PAD // NSUB

_mesh = plsc.VectorSubcoreMesh(core_axis_name="c", subcore_axis_name="s",
                               num_cores=NCORE, num_subcores=NSUB)


def _mm_body(x_ref, wl_ref, wr_ref, o_ref):
    x = x_ref[0]
    o_ref[0, 0] = jnp.dot(x, wl_ref[0], preferred_element_type=jnp.float32)
    o_ref[1, 0] = jnp.dot(x, wr_ref[0], preferred_element_type=jnp.float32)


def _per_etype_matmul(x3, wl, wr):
    bn = 2048
    return pl.pallas_call(
        _mm_body,
        grid=(NUM_ET, N_PAD // bn),
        in_specs=[
            pl.BlockSpec((1, bn, D),
                         lambda e, i: (jnp.where(
                             (e < 3) | (e == 8), 1,
                             jnp.where((e == 6) | (e == 9), 2, 0)), i, 0)),
            pl.BlockSpec((1, D, HD), lambda e, i: (e, 0, 0)),
            pl.BlockSpec((1, D, HD), lambda e, i: (e, 0, 0)),
        ],
        out_specs=pl.BlockSpec((2, 1, bn, HD), lambda e, i: (0, e, i, 0)),
        out_shape=jax.ShapeDtypeStruct((2, NUM_ET, N_PAD, HD), jnp.float32),
    )(x3, wl, wr)


def _segsum_body(with_counts, wh_hbm, src_hbm, dst_hbm, *refs):
    if with_counts:
        (sums_hbm, cnts_hbm, src_v, dst_v, rows0_v, rows1_v, rows2_v, ones_v,
         zrows_v, zcnt_v, acc_sh, cnt_sh, g0, g1, g2, s0, s1, s2, zsem) = refs
    else:
        (sums_hbm, src_v, dst_v, rows0_v, rows1_v, rows2_v, zrows_v, acc_sh,
         g0, g1, g2, s0, s1, s2, zsem) = refs
    rows = (rows0_v, rows1_v, rows2_v)
    sems = (g0, g1, g2)
    ssems = (s0, s1, s2)
    c = lax.axis_index("c")
    w = lax.axis_index("s")
    zero16 = jnp.zeros((16,), jnp.float32)
    one16 = jnp.ones((16,), jnp.float32)

    @pl.loop(0, CH)
    def _(r):
        for k in range(HD // 16):
            zrows_v[r, pl.ds(16 * k, 16)] = zero16
        if with_counts:
            zcnt_v[r, pl.ds(0, 16)] = zero16
            ones_v[r, pl.ds(0, 16)] = one16

    def start_gather(k, b):
        pltpu.async_copy(wh_hbm.at[src_v.at[k]], rows[b], sems[b])

    def wait_gather(k, b):
        pltpu.make_async_copy(wh_hbm.at[src_v.at[k]], rows[b], sems[b]).wait()

    def start_scatter(e, k, b):
        pltpu.async_copy(rows[b], acc_sh.at[dst_v.at[k]], ssems[b], add=True)
        if with_counts:
            @pl.when(c == e // 5)
            def _():
                pltpu.async_copy(ones_v, cnt_sh.at[dst_v.at[k]], ssems[b],
                                 add=True)

    def wait_scatter(e, k, b):
        pltpu.make_async_copy(rows[b], acc_sh.at[dst_v.at[k]],
                              ssems[b]).wait()
        if with_counts:
            @pl.when(c == e // 5)
            def _():
                pltpu.make_async_copy(ones_v, cnt_sh.at[dst_v.at[k]],
                                      ssems[b]).wait()

    @pl.loop(0, NUM_ET)
    def _(e):
        base_row = w * ROWS_PER_SUB

        @pl.loop(0, ROWS_PER_SUB // CH)
        def _(t):
            pltpu.async_copy(zrows_v, acc_sh.at[pl.ds(base_row + t * CH, CH)],
                             zsem)
            if with_counts:
                pltpu.async_copy(zcnt_v,
                                 cnt_sh.at[pl.ds(base_row + t * CH, CH)], zsem)

        pltpu.sync_copy(src_hbm.at[c, e, w], src_v)
        pltpu.sync_copy(dst_hbm.at[e, w], dst_v)

        @pl.loop(0, ROWS_PER_SUB // CH)
        def _(t):
            pltpu.make_async_copy(
                zrows_v, acc_sh.at[pl.ds(base_row + t * CH, CH)], zsem).wait()
            if with_counts:
                pltpu.make_async_copy(
                    zcnt_v, cnt_sh.at[pl.ds(base_row + t * CH, CH)],
                    zsem).wait()

        plsc.subcore_barrier()

        start_gather(0, 0)
        start_gather(1, 1)

        @pl.loop(0, (CPS - 1) // 3)
        def _(rr):
            for b in range(3):
                k = 3 * rr + b
                wait_gather(k, b)
                start_scatter(e, k, b)
                nb = (b + 2) % 3

                @pl.when(k >= 1)
                def _():
                    wait_scatter(e, k - 1, nb)

                @pl.when(k + 2 <= CPS - 1)
                def _():
                    start_gather(k + 2, nb)

        wait_gather(CPS - 1, (CPS - 1) % 3)
        start_scatter(e, CPS - 1, (CPS - 1) % 3)
        wait_scatter(e, CPS - 2, (CPS - 2) % 3)
        wait_scatter(e, CPS - 1, (CPS - 1) % 3)

        plsc.subcore_barrier()
        pltpu.sync_copy(acc_sh.at[pl.ds(base_row, ROWS_PER_SUB)],
                        sums_hbm.at[c, e, pl.ds(base_row, ROWS_PER_SUB)])
        if with_counts:
            @pl.when(c == e // 5)
            def _():
                pltpu.sync_copy(cnt_sh.at[pl.ds(base_row, ROWS_PER_SUB)],
                                cnts_hbm.at[e, pl.ds(base_row, ROWS_PER_SUB)])


def _sc_segsum(wh_flat, src_g2, dst_l, with_counts):
    outs = [jax.ShapeDtypeStruct((2, NUM_ET, N_PAD, HD), jnp.float32)]
    scratch = [
        pltpu.VMEM((CPS, CH), jnp.int32),
        pltpu.VMEM((CPS, CH), jnp.int32),
        pltpu.VMEM((CH, HD), jnp.float32),
        pltpu.VMEM((CH, HD), jnp.float32),
        pltpu.VMEM((CH, HD), jnp.float32),
    ]
    if with_counts:
        outs.append(jax.ShapeDtypeStruct((NUM_ET, N_PAD, 16), jnp.float32))
        scratch.append(pltpu.VMEM((CH, 16), jnp.float32))
    scratch.append(pltpu.VMEM((CH, HD), jnp.float32))
    if with_counts:
        scratch.append(pltpu.VMEM((CH, 16), jnp.float32))
    scratch.append(pltpu.VMEM_SHARED((N_PAD, HD), jnp.float32))
    if with_counts:
        scratch.append(pltpu.VMEM_SHARED((N_PAD, 16), jnp.float32))
    for _ in range(7):
        scratch.append(pltpu.SemaphoreType.DMA)

    k = pl.kernel(
        functools.partial(_segsum_body, with_counts),
        out_type=tuple(outs),
        mesh=_mesh,
        scratch_types=scratch,
        compiler_params=pltpu.CompilerParams(use_tc_tiling_on_sc=False),
    )
    return k(wh_flat, src_g2, dst_l)


_G0 = [e for e in range(NUM_ET) if _DST[e] == 0]
_G1 = [e for e in range(NUM_ET) if _DST[e] == 1]
_G2 = [e for e in range(NUM_ET) if _DST[e] == 2]


def _mean_reduce(s_ref, c_ref):
    cnt = jnp.maximum(c_ref[:, :, 0:1], 1.0)
    halves = []
    for h in range(2):
        m = s_ref[h] / cnt
        hs = []
        for grp in (_G0, _G1, _G2):
            acc = m[grp[0]]
            for e in grp[1:]:
                acc = acc + m[e]
            hs.append(acc)
        halves.append(jnp.stack(hs, axis=0))
    return jnp.concatenate(halves, axis=-1)


def _mean_body(s_ref, c_ref, o_ref):
    out = _mean_reduce(s_ref, c_ref)
    o_ref[...] = jnp.where(out >= 0.0, out, 0.01 * out)


def _mean_stage(sums, cnts):
    bn = 1024
    return pl.pallas_call(
        _mean_body,
        grid=(N_PAD // bn,),
        in_specs=[
            pl.BlockSpec((2, NUM_ET, bn, HD), lambda i: (0, 0, i, 0)),
            pl.BlockSpec((NUM_ET, bn, 16), lambda i: (0, i, 0)),
        ],
        out_specs=pl.BlockSpec((3, bn, D), lambda i: (0, i, 0)),
        out_shape=jax.ShapeDtypeStruct((3, N_PAD, D), jnp.float32),
    )(sums, cnts)


def _pack_bf16_pair(x):
    lo = lax.bitcast_convert_type(
        x[..., :HD].astype(jnp.bfloat16), jnp.uint16).astype(jnp.uint32)
    hi = lax.bitcast_convert_type(
        x[..., HD:].astype(jnp.bfloat16), jnp.uint16).astype(jnp.uint32)
    return lax.bitcast_convert_type(lo | (hi << 16), jnp.int32)


def _mean2_body(s_ref, c_ref, w_ref, h_ref, g_ref):
    out = _mean_reduce(s_ref, c_ref)
    w = w_ref[...]
    h_ref[...] = _pack_bf16_pair(out)
    g_ref[...] = _pack_bf16_pair(jnp.stack(
        [out[_DST16[s]] * w[_ET16[s]][None, :] for s in range(16)], axis=0))


def _mean2_pack(sums, cnts, w_rels):
    bn = 512
    return pl.pallas_call(
        _mean2_body,
        grid=(N_PAD // bn,),
        in_specs=[
            pl.BlockSpec((2, NUM_ET, bn, HD), lambda i: (0, 0, i, 0)),
            pl.BlockSpec((NUM_ET, bn, 16), lambda i: (0, i, 0)),
            pl.BlockSpec((NUM_ET, D), lambda i: (0, 0)),
        ],
        out_specs=[
            pl.BlockSpec((3, bn, HD), lambda i: (0, i, 0)),
            pl.BlockSpec((16, bn, HD), lambda i: (0, i, 0)),
        ],
        out_shape=[
            jax.ShapeDtypeStruct((3, N_PAD, HD), jnp.int32),
            jax.ShapeDtypeStruct((16, N_PAD, HD), jnp.int32),
        ],
    )(sums, cnts, w_rels)


def _distmult_body(h_hbm, g_hbm, src_hbm, dst_hbm, p_hbm,
                   src_v, dst_v, u0_v, u1_v, g0_v, g1_v, p_v,
                   semu0, semu1, semg0, semg1):
    c = lax.axis_index("c")
    w = lax.axis_index("s")
    us = (u0_v, u1_v)
    gs = (g0_v, g1_v)
    semus = (semu0, semu1)
    semgs = (semg0, semg1)

    def start_gathers(k, b):
        pltpu.async_copy(h_hbm.at[src_v.at[k]], us[b], semus[b])
        pltpu.async_copy(g_hbm.at[dst_v.at[k]], gs[b], semgs[b])

    def compute_chunk(k, b):
        pltpu.make_async_copy(h_hbm.at[src_v.at[k]], us[b], semus[b]).wait()
        pltpu.make_async_copy(g_hbm.at[dst_v.at[k]], gs[b], semgs[b]).wait()
        u_v, g_v = us[b], gs[b]

        @pl.loop(0, CH // 8)
        def _(r8):
            for i in range(8):
                r = r8 * 8 + i
                acc = None
                for q in range(4):
                    uq = plsc.bitcast(u_v[r, pl.ds(16 * q, 16)], jnp.bfloat16)
                    gq = plsc.bitcast(g_v[r, pl.ds(16 * q, 16)], jnp.bfloat16)
                    t = uq * gq
                    acc = t if acc is None else acc + t
                pa, pb = plsc.unpack(acc, format=plsc.PackFormat.INTERLEAVED)
                p_v[k, r8, pl.ds(16 * i, 16)] = pa + pb

    @pl.loop(0, 8)
    def _(j):
        s = c * 8 + j
        pltpu.sync_copy(src_hbm.at[s, w], src_v)
        pltpu.sync_copy(dst_hbm.at[s, w], dst_v)

        start_gathers(0, 0)

        @pl.loop(0, (CPS - 1) // 2)
        def _(kk):
            k = 2 * kk
            start_gathers(k + 1, 1)
            compute_chunk(k, 0)
            start_gathers(k + 2, 0)
            compute_chunk(k + 1, 1)

        compute_chunk(CPS - 1, 0)

        pltpu.sync_copy(p_v, p_hbm.at[s, w])


def _sc_distmult(h2_flat, g_flat, src16, dst16):
    k = pl.kernel(
        _distmult_body,
        out_type=jax.ShapeDtypeStruct((16, NSUB, CPS, CH // 8, 128),
                                      jnp.float32),
        mesh=_mesh,
        scratch_types=[
            pltpu.VMEM((CPS, CH), jnp.int32),
            pltpu.VMEM((CPS, CH), jnp.int32),
            pltpu.VMEM((CH, HD), jnp.int32),
            pltpu.VMEM((CH, HD), jnp.int32),
            pltpu.VMEM((CH, HD), jnp.int32),
            pltpu.VMEM((CH, HD), jnp.int32),
            pltpu.VMEM((CPS, CH // 8, 128), jnp.float32),
            pltpu.SemaphoreType.DMA,
            pltpu.SemaphoreType.DMA,
            pltpu.SemaphoreType.DMA,
            pltpu.SemaphoreType.DMA,
        ],
        compiler_params=pltpu.CompilerParams(needs_layout_passes=False,
                                             use_tc_tiling_on_sc=False),
    )
    return k(h2_flat, g_flat, src16, dst16)


def _finish_body(p_ref, s_ref, o_ref):
    o_ref[0] = jnp.dot(p_ref[0], s_ref[...],
                       preferred_element_type=jnp.float32)


def _finish(p16):
    sel = np.zeros((128, 8), np.float32)
    for d in range(128):
        sel[d, d // 16] = 1.0
    sel = jnp.asarray(sel)
    bn = 6400
    return pl.pallas_call(
        _finish_body,
        grid=(16, (E_PAD // 8) // bn),
        in_specs=[
            pl.BlockSpec((1, bn, 128), lambda s, i: (s, i, 0)),
            pl.BlockSpec((128, 8), lambda s, i: (0, 0)),
        ],
        out_specs=pl.BlockSpec((1, bn, 8), lambda s, i: (s, i, 0)),
        out_shape=jax.ShapeDtypeStruct((16, E_PAD // 8, 8), jnp.float32),
    )(p16, sel)


def kernel(x_disease, x_drug, x_gene, W1, W2, w_rels, edges_all, neg_edges):
    x3 = jnp.stack([x_disease, x_drug, x_gene], axis=0)
    x3 = jnp.pad(x3, ((0, 0), (0, N_PAD - N), (0, 0)))

    src_l = jnp.pad(edges_all[:, 0, :], ((0, 0), (0, E_PAD - E)))
    dst_l = jnp.pad(edges_all[:, 1, :], ((0, 0), (0, E_PAD - E)),
                    constant_values=N)
    et_off = (jnp.arange(NUM_ET, dtype=jnp.int32) * N_PAD)[:, None]
    src_g = src_l + et_off
    src_g2 = jnp.stack([src_g, src_g + NUM_ET * N_PAD], axis=0
                       ).reshape(2, NUM_ET, NSUB, CPS, CH)
    dst_l = dst_l.reshape(NUM_ET, NSUB, CPS, CH)

    s16 = jnp.concatenate([edges_all[:, 0, :], neg_edges[:, 0, :]], axis=0)
    d16 = jnp.concatenate([edges_all[:, 1, :], neg_edges[:, 1, :]], axis=0)
    s16 = jnp.pad(s16, ((0, 0), (0, E_PAD - E)))
    d16 = jnp.pad(d16, ((0, 0), (0, E_PAD - E)))
    src16 = (s16 + (jnp.asarray(_SRC16, jnp.int32) * N_PAD)[:, None]
             ).reshape(16, NSUB, CPS, CH)
    dst16 = (d16 + (jnp.arange(16, dtype=jnp.int32) * N_PAD)[:, None]
             ).reshape(16, NSUB, CPS, CH)

    wh1 = _per_etype_matmul(x3, W1[:, :, :HD], W1[:, :, HD:]
                            ).reshape(2 * NUM_ET * N_PAD, HD)
    sums1, cnts = _sc_segsum(wh1, src_g2, dst_l, with_counts=True)
    h1 = _mean_stage(sums1, cnts)

    wh2 = _per_etype_matmul(h1, W2[:, :, :HD], W2[:, :, HD:]
                            ).reshape(2 * NUM_ET * N_PAD, HD)
    (sums2,) = _sc_segsum(wh2, src_g2, dst_l, with_counts=False)

    h2p, g16p = _mean2_pack(sums2, cnts, w_rels)
    p16 = _sc_distmult(h2p.reshape(3 * N_PAD, HD),
                       g16p.reshape(16 * N_PAD, HD), src16, dst16)
    scores = _finish(p16.reshape(16, E_PAD // 8, 128))
    return scores.reshape(16, E_PAD)[:, :E].reshape(-1)


# speedup vs baseline: 1.2879x; 1.0121x over previous
import functools

import jax
import jax.numpy as jnp
import numpy as np
from jax import lax
from jax.experimental import pallas as pl
from jax.experimental.pallas import tpu as pltpu
from jax.experimental.pallas import tpu_sc as plsc

N = 10000
D = 128
HD = D // 2
E = 50000
NUM_ET = 10

_SRC = [1, 1, 1, 0, 0, 0, 2, 0, 1, 2]
_DST = [0, 0, 0, 1, 1, 1, 2, 0, 2, 0]

_ET16 = list(range(10)) + list(range(6))
_SRC16 = [_SRC[e] for e in _ET16]
_DST16 = [_DST[e] for e in _ET16]

N_PAD = 10240
E_PAD = 51200
CH = 128
CHUNKS = E_PAD // CH
NSUB = 16
NCORE = 2
CPS = CHUNKS // NSUB
ROWS_PER_SUB = N_PAD // NSUB

_mesh = plsc.VectorSubcoreMesh(core_axis_name="c", subcore_axis_name="s",
                               num_cores=NCORE, num_subcores=NSUB)


def _mm_body(x_ref, wl_ref, wr_ref, o_ref):
    x = x_ref[0]
    o_ref[0, 0] = jnp.dot(x, wl_ref[0], preferred_element_type=jnp.float32)
    o_ref[1, 0] = jnp.dot(x, wr_ref[0], preferred_element_type=jnp.float32)


def _per_etype_matmul(x3, wl, wr):
    bn = 5120
    return pl.pallas_call(
        _mm_body,
        grid=(NUM_ET, N_PAD // bn),
        in_specs=[
            pl.BlockSpec((1, bn, D),
                         lambda e, i: (jnp.where(
                             (e < 3) | (e == 8), 1,
                             jnp.where((e == 6) | (e == 9), 2, 0)), i, 0)),
            pl.BlockSpec((1, D, HD), lambda e, i: (e, 0, 0)),
            pl.BlockSpec((1, D, HD), lambda e, i: (e, 0, 0)),
        ],
        out_specs=pl.BlockSpec((2, 1, bn, HD), lambda e, i: (0, e, i, 0)),
        out_shape=jax.ShapeDtypeStruct((2, NUM_ET, N_PAD, HD), jnp.float32),
    )(x3, wl, wr)


def _segsum_body(with_counts, wh_hbm, src_hbm, dst_hbm, *refs):
    if with_counts:
        (sums_hbm, cnts_hbm, src_v, dst_v, rows0_v, rows1_v, rows2_v, ones_v,
         zrows_v, zcnt_v, acc_sh, cnt_sh, g0, g1, g2, s0, s1, s2, zsem) = refs
    else:
        (sums_hbm, src_v, dst_v, rows0_v, rows1_v, rows2_v, zrows_v, acc_sh,
         g0, g1, g2, s0, s1, s2, zsem) = refs
    rows = (rows0_v, rows1_v, rows2_v)
    sems = (g0, g1, g2)
    ssems = (s0, s1, s2)
    c = lax.axis_index("c")
    w = lax.axis_index("s")
    zero16 = jnp.zeros((16,), jnp.float32)
    one16 = jnp.ones((16,), jnp.float32)

    @pl.loop(0, CH)
    def _(r):
        for k in range(HD // 16):
            zrows_v[r, pl.ds(16 * k, 16)] = zero16
        if with_counts:
            zcnt_v[r, pl.ds(0, 16)] = zero16
            ones_v[r, pl.ds(0, 16)] = one16

    def start_gather(k, b):
        pltpu.async_copy(wh_hbm.at[src_v.at[k]], rows[b], sems[b])

    def wait_gather(k, b):
        pltpu.make_async_copy(wh_hbm.at[src_v.at[k]], rows[b], sems[b]).wait()

    def start_scatter(e, k, b):
        pltpu.async_copy(rows[b], acc_sh.at[dst_v.at[k]], ssems[b], add=True)
        if with_counts:
            @pl.when(c == e // 5)
            def _():
                pltpu.async_copy(ones_v, cnt_sh.at[dst_v.at[k]], ssems[b],
                                 add=True)

    def wait_scatter(e, k, b):
        pltpu.make_async_copy(rows[b], acc_sh.at[dst_v.at[k]],
                              ssems[b]).wait()
        if with_counts:
            @pl.when(c == e // 5)
            def _():
                pltpu.make_async_copy(ones_v, cnt_sh.at[dst_v.at[k]],
                                      ssems[b]).wait()

    @pl.loop(0, NUM_ET)
    def _(e):
        base_row = w * ROWS_PER_SUB

        @pl.loop(0, ROWS_PER_SUB // CH)
        def _(t):
            pltpu.async_copy(zrows_v, acc_sh.at[pl.ds(base_row + t * CH, CH)],
                             zsem)
            if with_counts:
                pltpu.async_copy(zcnt_v,
                                 cnt_sh.at[pl.ds(base_row + t * CH, CH)], zsem)

        pltpu.sync_copy(src_hbm.at[c, e, w], src_v)
        pltpu.sync_copy(dst_hbm.at[e, w], dst_v)

        @pl.loop(0, ROWS_PER_SUB // CH)
        def _(t):
            pltpu.make_async_copy(
                zrows_v, acc_sh.at[pl.ds(base_row + t * CH, CH)], zsem).wait()
            if with_counts:
                pltpu.make_async_copy(
                    zcnt_v, cnt_sh.at[pl.ds(base_row + t * CH, CH)],
                    zsem).wait()

        plsc.subcore_barrier()

        start_gather(0, 0)
        start_gather(1, 1)

        @pl.loop(0, (CPS - 1) // 3)
        def _(rr):
            for b in range(3):
                k = 3 * rr + b
                wait_gather(k, b)
                start_scatter(e, k, b)
                nb = (b + 2) % 3

                @pl.when(k >= 1)
                def _():
                    wait_scatter(e, k - 1, nb)

                @pl.when(k + 2 <= CPS - 1)
                def _():
                    start_gather(k + 2, nb)

        wait_gather(CPS - 1, (CPS - 1) % 3)
        start_scatter(e, CPS - 1, (CPS - 1) % 3)
        wait_scatter(e, CPS - 2, (CPS - 2) % 3)
        wait_scatter(e, CPS - 1, (CPS - 1) % 3)

        plsc.subcore_barrier()
        pltpu.sync_copy(acc_sh.at[pl.ds(base_row, ROWS_PER_SUB)],
                        sums_hbm.at[c, e, pl.ds(base_row, ROWS_PER_SUB)])
        if with_counts:
            @pl.when(c == e // 5)
            def _():
                pltpu.sync_copy(cnt_sh.at[pl.ds(base_row, ROWS_PER_SUB)],
                                cnts_hbm.at[e, pl.ds(base_row, ROWS_PER_SUB)])


def _sc_segsum(wh_flat, src_g2, dst_l, with_counts):
    outs = [jax.ShapeDtypeStruct((2, NUM_ET, N_PAD, HD), jnp.float32)]
    scratch = [
        pltpu.VMEM((CPS, CH), jnp.int32),
        pltpu.VMEM((CPS, CH), jnp.int32),
        pltpu.VMEM((CH, HD), jnp.float32),
        pltpu.VMEM((CH, HD), jnp.float32),
        pltpu.VMEM((CH, HD), jnp.float32),
    ]
    if with_counts:
        outs.append(jax.ShapeDtypeStruct((NUM_ET, N_PAD, 16), jnp.float32))
        scratch.append(pltpu.VMEM((CH, 16), jnp.float32))
    scratch.append(pltpu.VMEM((CH, HD), jnp.float32))
    if with_counts:
        scratch.append(pltpu.VMEM((CH, 16), jnp.float32))
    scratch.append(pltpu.VMEM_SHARED((N_PAD, HD), jnp.float32))
    if with_counts:
        scratch.append(pltpu.VMEM_SHARED((N_PAD, 16), jnp.float32))
    for _ in range(7):
        scratch.append(pltpu.SemaphoreType.DMA)

    k = pl.kernel(
        functools.partial(_segsum_body, with_counts),
        out_type=tuple(outs),
        mesh=_mesh,
        scratch_types=scratch,
        compiler_params=pltpu.CompilerParams(use_tc_tiling_on_sc=False),
    )
    return k(wh_flat, src_g2, dst_l)


_G0 = [e for e in range(NUM_ET) if _DST[e] == 0]
_G1 = [e for e in range(NUM_ET) if _DST[e] == 1]
_G2 = [e for e in range(NUM_ET) if _DST[e] == 2]


def _mean_reduce(s_ref, c_ref):
    cnt = jnp.maximum(c_ref[:, :, 0:1], 1.0)
    halves = []
    for h in range(2):
        m = s_ref[h] / cnt
        hs = []
        for grp in (_G0, _G1, _G2):
            acc = m[grp[0]]
            for e in grp[1:]:
                acc = acc + m[e]
            hs.append(acc)
        halves.append(jnp.stack(hs, axis=0))
    return jnp.concatenate(halves, axis=-1)


def _mean_body(s_ref, c_ref, o_ref):
    out = _mean_reduce(s_ref, c_ref)
    o_ref[...] = jnp.where(out >= 0.0, out, 0.01 * out)


def _mean_stage(sums, cnts):
    bn = 1024
    return pl.pallas_call(
        _mean_body,
        grid=(N_PAD // bn,),
        in_specs=[
            pl.BlockSpec((2, NUM_ET, bn, HD), lambda i: (0, 0, i, 0)),
            pl.BlockSpec((NUM_ET, bn, 16), lambda i: (0, i, 0)),
        ],
        out_specs=pl.BlockSpec((3, bn, D), lambda i: (0, i, 0)),
        out_shape=jax.ShapeDtypeStruct((3, N_PAD, D), jnp.float32),
    )(sums, cnts)


def _pack_bf16_pair(x):
    lo = lax.bitcast_convert_type(
        x[..., :HD].astype(jnp.bfloat16), jnp.uint16).astype(jnp.uint32)
    hi = lax.bitcast_convert_type(
        x[..., HD:].astype(jnp.bfloat16), jnp.uint16).astype(jnp.uint32)
    return lax.bitcast_convert_type(lo | (hi << 16), jnp.int32)


def _mean2_body(s_ref, c_ref, w_ref, h_ref, g_ref):
    out = _mean_reduce(s_ref, c_ref)
    w = w_ref[...]
    h_ref[...] = _pack_bf16_pair(out)
    g_ref[...] = _pack_bf16_pair(jnp.stack(
        [out[_DST16[s]] * w[_ET16[s]][None, :] for s in range(16)], axis=0))


def _mean2_pack(sums, cnts, w_rels):
    bn = 512
    return pl.pallas_call(
        _mean2_body,
        grid=(N_PAD // bn,),
        in_specs=[
            pl.BlockSpec((2, NUM_ET, bn, HD), lambda i: (0, 0, i, 0)),
            pl.BlockSpec((NUM_ET, bn, 16), lambda i: (0, i, 0)),
            pl.BlockSpec((NUM_ET, D), lambda i: (0, 0)),
        ],
        out_specs=[
            pl.BlockSpec((3, bn, HD), lambda i: (0, i, 0)),
            pl.BlockSpec((16, bn, HD), lambda i: (0, i, 0)),
        ],
        out_shape=[
            jax.ShapeDtypeStruct((3, N_PAD, HD), jnp.int32),
            jax.ShapeDtypeStruct((16, N_PAD, HD), jnp.int32),
        ],
    )(sums, cnts, w_rels)


def _distmult_body(h_hbm, g_hbm, src_hbm, dst_hbm, p_hbm,
                   src_v, dst_v, u0_v, u1_v, g0_v, g1_v, p_v,
                   semu0, semu1, semg0, semg1):
    c = lax.axis_index("c")
    w = lax.axis_index("s")
    us = (u0_v, u1_v)
    gs = (g0_v, g1_v)
    semus = (semu0, semu1)
    semgs = (semg0, semg1)

    def start_gathers(k, b):
        pltpu.async_copy(h_hbm.at[src_v.at[k]], us[b], semus[b])
        pltpu.async_copy(g_hbm.at[dst_v.at[k]], gs[b], semgs[b])

    def compute_chunk(k, b):
        pltpu.make_async_copy(h_hbm.at[src_v.at[k]], us[b], semus[b]).wait()
        pltpu.make_async_copy(g_hbm.at[dst_v.at[k]], gs[b], semgs[b]).wait()
        u_v, g_v = us[b], gs[b]

        @pl.loop(0, CH // 8)
        def _(r8):
            for i in range(8):
                r = r8 * 8 + i
                acc = None
                for q in range(4):
                    uq = plsc.bitcast(u_v[r, pl.ds(16 * q, 16)], jnp.bfloat16)
                    gq = plsc.bitcast(g_v[r, pl.ds(16 * q, 16)], jnp.bfloat16)
                    t = uq * gq
                    acc = t if acc is None else acc + t
                pa, pb = plsc.unpack(acc, format=plsc.PackFormat.INTERLEAVED)
                p_v[k, r8, pl.ds(16 * i, 16)] = pa + pb

    @pl.loop(0, 8)
    def _(j):
        s = c * 8 + j
        pltpu.sync_copy(src_hbm.at[s, w], src_v)
        pltpu.sync_copy(dst_hbm.at[s, w], dst_v)

        start_gathers(0, 0)

        @pl.loop(0, (CPS - 1) // 2)
        def _(kk):
            k = 2 * kk
            start_gathers(k + 1, 1)
            compute_chunk(k, 0)
            start_gathers(k + 2, 0)
            compute_chunk(k + 1, 1)

        compute_chunk(CPS - 1, 0)

        pltpu.sync_copy(p_v, p_hbm.at[s, w])


def _sc_distmult(h2_flat, g_flat, src16, dst16):
    k = pl.kernel(
        _distmult_body,
        out_type=jax.ShapeDtypeStruct((16, NSUB, CPS, CH // 8, 128),
                                      jnp.float32),
        mesh=_mesh,
        scratch_types=[
            pltpu.VMEM((CPS, CH), jnp.int32),
            pltpu.VMEM((CPS, CH), jnp.int32),
            pltpu.VMEM((CH, HD), jnp.int32),
            pltpu.VMEM((CH, HD), jnp.int32),
            pltpu.VMEM((CH, HD), jnp.int32),
            pltpu.VMEM((CH, HD), jnp.int32),
            pltpu.VMEM((CPS, CH // 8, 128), jnp.float32),
            pltpu.SemaphoreType.DMA,
            pltpu.SemaphoreType.DMA,
            pltpu.SemaphoreType.DMA,
            pltpu.SemaphoreType.DMA,
        ],
        compiler_params=pltpu.CompilerParams(needs_layout_passes=False,
                                             use_tc_tiling_on_sc=False),
    )
    return k(h2_flat, g_flat, src16, dst16)


def _finish_body(p_ref, s_ref, o_ref):
    o_ref[0] = jnp.dot(p_ref[0], s_ref[...],
                       preferred_element_type=jnp.float32)


def _finish(p16):
    sel = np.zeros((128, 8), np.float32)
    for d in range(128):
        sel[d, d // 16] = 1.0
    sel = jnp.asarray(sel)
    bn = 6400
    return pl.pallas_call(
        _finish_body,
        grid=(16, (E_PAD // 8) // bn),
        in_specs=[
            pl.BlockSpec((1, bn, 128), lambda s, i: (s, i, 0)),
            pl.BlockSpec((128, 8), lambda s, i: (0, 0)),
        ],
        out_specs=pl.BlockSpec((1, bn, 8), lambda s, i: (s, i, 0)),
        out_shape=jax.ShapeDtypeStruct((16, E_PAD // 8, 8), jnp.float32),
    )(p16, sel)


def kernel(x_disease, x_drug, x_gene, W1, W2, w_rels, edges_all, neg_edges):
    x3 = jnp.stack([x_disease, x_drug, x_gene], axis=0)
    x3 = jnp.pad(x3, ((0, 0), (0, N_PAD - N), (0, 0)))

    src_l = jnp.pad(edges_all[:, 0, :], ((0, 0), (0, E_PAD - E)))
    dst_l = jnp.pad(edges_all[:, 1, :], ((0, 0), (0, E_PAD - E)),
                    constant_values=N)
    et_off = (jnp.arange(NUM_ET, dtype=jnp.int32) * N_PAD)[:, None]
    src_g = src_l + et_off
    src_g2 = jnp.stack([src_g, src_g + NUM_ET * N_PAD], axis=0
                       ).reshape(2, NUM_ET, NSUB, CPS, CH)
    dst_l = dst_l.reshape(NUM_ET, NSUB, CPS, CH)

    s16 = jnp.concatenate([edges_all[:, 0, :], neg_edges[:, 0, :]], axis=0)
    d16 = jnp.concatenate([edges_all[:, 1, :], neg_edges[:, 1, :]], axis=0)
    s16 = jnp.pad(s16, ((0, 0), (0, E_PAD - E)))
    d16 = jnp.pad(d16, ((0, 0), (0, E_PAD - E)))
    src16 = (s16 + (jnp.asarray(_SRC16, jnp.int32) * N_PAD)[:, None]
             ).reshape(16, NSUB, CPS, CH)
    dst16 = (d16 + (jnp.arange(16, dtype=jnp.int32) * N_PAD)[:, None]
             ).reshape(16, NSUB, CPS, CH)

    wh1 = _per_etype_matmul(x3, W1[:, :, :HD], W1[:, :, HD:]
                            ).reshape(2 * NUM_ET * N_PAD, HD)
    sums1, cnts = _sc_segsum(wh1, src_g2, dst_l, with_counts=True)
    h1 = _mean_stage(sums1, cnts)

    wh2 = _per_etype_matmul(h1, W2[:, :, :HD], W2[:, :, HD:]
                            ).reshape(2 * NUM_ET * N_PAD, HD)
    (sums2,) = _sc_segsum(wh2, src_g2, dst_l, with_counts=False)

    h2p, g16p = _mean2_pack(sums2, cnts, w_rels)
    p16 = _sc_distmult(h2p.reshape(3 * N_PAD, HD),
                       g16p.reshape(16 * N_PAD, HD), src16, dst16)
    scores = _finish(p16.reshape(16, E_PAD // 8, 128))
    return scores.reshape(16, E_PAD)[:, :E].reshape(-1)
